# baseline (speedup 1.0000x reference)
import jax
import jax.numpy as jnp
from jax import lax
from jax.experimental import pallas as pl
from jax.experimental.pallas import tpu as pltpu
from jax.experimental.pallas import tpu_sc as plsc

N = 27554
K = 9
NLANE = 16
NTILES = 32
CHUNK = 864
VLAST = N - (NTILES - 1) * CHUNK
GROUPS = CHUNK // NLANE
DP = 27560
CP = 872
INV_COUNT = 1.0 / (3.0 * N)

DELTA_D = [(c * N) % 8 for c in range(3)]
DELTA_I = [(j * N) % 8 for j in range(K)]


def _final_body(p_ref, o_ref):
    o_ref[...] = (jnp.sum(p_ref[...]) * INV_COUNT).reshape(1, 1)


def _sc_body(d_hbm, idx_hbm, aw_hbm, lw_hbm, out_hbm,
             d_v, idx_v, aw_v, lw_v, acc_v, sem):
    cid = lax.axis_index("c")
    sid = lax.axis_index("s")
    wid = sid * 2 + cid
    base = wid * CHUNK

    d_copies = []
    for c in range(3):
        src = c * N - DELTA_D[c]
        d_copies.append(pltpu.async_copy(
            d_hbm.at[pl.ds(src, N + DELTA_D[c])],
            d_v.at[pl.ds(c * DP, N + DELTA_D[c])], sem))

    @pl.when(wid != NTILES - 1)
    def _full_chunk():
        copies = []
        for j in range(K):
            src = j * N + base - DELTA_I[j]
            copies.append(pltpu.async_copy(
                idx_hbm.at[pl.ds(src, CHUNK + DELTA_I[j])],
                idx_v.at[pl.ds(j * CP, CHUNK + DELTA_I[j])], sem))
        copies.append(pltpu.async_copy(
            aw_hbm.at[pl.ds(base, CHUNK)], aw_v, sem))
        copies.append(pltpu.async_copy(
            lw_hbm.at[pl.ds(base, CHUNK)], lw_v, sem))
        for cp in copies:
            cp.wait()

    @pl.when(wid == NTILES - 1)
    def _tail_chunk():
        copies = []
        for j in range(K):
            src = j * N + base - DELTA_I[j]
            copies.append(pltpu.async_copy(
                idx_hbm.at[pl.ds(src, VLAST + DELTA_I[j])],
                idx_v.at[pl.ds(j * CP, VLAST + DELTA_I[j])], sem))
        copies.append(pltpu.async_copy(
            aw_hbm.at[pl.ds(base, VLAST)], aw_v.at[pl.ds(0, VLAST)], sem))
        copies.append(pltpu.async_copy(
            lw_hbm.at[pl.ds(base, VLAST)], lw_v.at[pl.ds(0, VLAST)], sem))
        for cp in copies:
            cp.wait()

    for cp in d_copies:
        cp.wait()

    iota = lax.iota(jnp.int32, NLANE)
    nmax = jnp.full((NLANE,), N - 1, jnp.int32)
    zero = jnp.zeros((NLANE,), jnp.int32)
    pc = [c * DP + DELTA_D[c] for c in range(3)]

    def group(g, acc):
        vb = g * NLANE
        vglob = base + vb + iota
        vmin = jnp.minimum(vglob, nmax)
        s0 = plsc.load_gather(d_v, [vmin + pc[0]])
        s1 = plsc.load_gather(d_v, [vmin + pc[1]])
        s2 = plsc.load_gather(d_v, [vmin + pc[2]])
        a0 = jnp.zeros((NLANE,), jnp.float32)
        a1 = jnp.zeros((NLANE,), jnp.float32)
        a2 = jnp.zeros((NLANE,), jnp.float32)
        for j in range(K):
            nb = plsc.load_gather(idx_v, [vb + iota + (j * CP + DELTA_I[j])])
            nb = jnp.minimum(jnp.maximum(nb, zero), nmax)
            a0 = a0 + plsc.load_gather(d_v, [nb + pc[0]])
            a1 = a1 + plsc.load_gather(d_v, [nb + pc[1]])
            a2 = a2 + plsc.load_gather(d_v, [nb + pc[2]])
        rw = 1.0 / aw_v[pl.ds(vb, NLANE)]
        r0 = s0 - a0 * rw
        r1 = s1 - a1 * rw
        r2 = s2 - a2 * rw
        lwt = lw_v[pl.ds(vb, NLANE)]
        term = (r0 * r0 + r1 * r1 + r2 * r2) * lwt
        term = jnp.where(vglob < N, term, 0.0)
        return acc + term

    acc = lax.fori_loop(0, GROUPS, group, jnp.zeros((NLANE,), jnp.float32))
    acc_v[...] = acc
    pltpu.sync_copy(acc_v, out_hbm.at[pl.ds(wid * NLANE, NLANE)])


_sc_call = pl.kernel(
    _sc_body,
    out_type=jax.ShapeDtypeStruct((NTILES * NLANE,), jnp.float32),
    mesh=plsc.VectorSubcoreMesh(core_axis_name="c", subcore_axis_name="s"),
    compiler_params=pltpu.CompilerParams(
        needs_layout_passes=False, use_tc_tiling_on_sc=False),
    scratch_types=[
        pltpu.VMEM((3 * DP,), jnp.float32),
        pltpu.VMEM((K * CP,), jnp.int32),
        pltpu.VMEM((CHUNK,), jnp.float32),
        pltpu.VMEM((CHUNK,), jnp.float32),
        pltpu.VMEM((NLANE,), jnp.float32),
        pltpu.SemaphoreType.DMA,
    ],
)


def kernel(v_1, v_2, adjacency_idx, adjacency_w, laplace_w):
    d_flat = (v_1 - v_2).T.reshape(3 * N)
    idx_flat = adjacency_idx.astype(jnp.int32).T.reshape(K * N)
    partials = _sc_call(
        d_flat, idx_flat, adjacency_w.reshape(N), laplace_w.reshape(N)
    ).reshape(NTILES, NLANE)
    out = pl.pallas_call(
        _final_body,
        out_shape=jax.ShapeDtypeStruct((1, 1), jnp.float32),
    )(partials)
    return out.reshape(())


# speedup vs baseline: 1.1648x; 1.1648x over previous
import jax
import jax.numpy as jnp
from jax import lax
from jax.experimental import pallas as pl
from jax.experimental.pallas import tpu as pltpu
from jax.experimental.pallas import tpu_sc as plsc

N = 27554
K = 9
NLANE = 16
NTILES = 32
CHUNK = 864
VLAST = N - (NTILES - 1) * CHUNK
GROUPS = CHUNK // NLANE
DP = 27568
BP = 27552
CP = 872
INV_COUNT = 1.0 / (3.0 * N)

DELTA_D = [(c * N) % 8 for c in range(3)]
DELTA_I = [(j * N) % 8 for j in range(K)]


def _final_body(p_ref, o_ref):
    o_ref[...] = (jnp.sum(p_ref[...]) * INV_COUNT).reshape(1, 1)


def _sc_body(d_hbm, idx_hbm, aw_hbm, lw_hbm, out_hbm,
             d_v, idx_v, aw_v, lw_v, acc_v, d_sp, sem):
    cid = lax.axis_index("c")
    sid = lax.axis_index("s")
    wid = sid * 2 + cid
    base = wid * CHUNK

    @pl.when(sid == 0)
    def _stage_d():
        copies = []
        for c in range(3):
            src = c * N - DELTA_D[c]
            copies.append(pltpu.async_copy(
                d_hbm.at[pl.ds(src, BP)],
                d_sp.at[pl.ds(c * BP, BP)], sem))
        for cp in copies:
            cp.wait()

    tail_copies = []
    for c in range(3):
        src = c * N - DELTA_D[c] + BP
        tail_copies.append(pltpu.async_copy(
            d_hbm.at[pl.ds(src, 2 + DELTA_D[c])],
            d_v.at[pl.ds(c * DP + BP, 2 + DELTA_D[c])], sem))

    @pl.when(wid != NTILES - 1)
    def _full_chunk():
        copies = []
        for j in range(K):
            src = j * N + base - DELTA_I[j]
            copies.append(pltpu.async_copy(
                idx_hbm.at[pl.ds(src, CHUNK + DELTA_I[j])],
                idx_v.at[pl.ds(j * CP, CHUNK + DELTA_I[j])], sem))
        copies.append(pltpu.async_copy(
            aw_hbm.at[pl.ds(base, CHUNK)], aw_v, sem))
        copies.append(pltpu.async_copy(
            lw_hbm.at[pl.ds(base, CHUNK)], lw_v, sem))
        for cp in copies:
            cp.wait()

    @pl.when(wid == NTILES - 1)
    def _tail_chunk():
        copies = []
        for j in range(K):
            src = j * N + base - DELTA_I[j]
            copies.append(pltpu.async_copy(
                idx_hbm.at[pl.ds(src, VLAST + DELTA_I[j])],
                idx_v.at[pl.ds(j * CP, VLAST + DELTA_I[j])], sem))
        copies.append(pltpu.async_copy(
            aw_hbm.at[pl.ds(base, VLAST)], aw_v.at[pl.ds(0, VLAST)], sem))
        copies.append(pltpu.async_copy(
            lw_hbm.at[pl.ds(base, VLAST)], lw_v.at[pl.ds(0, VLAST)], sem))
        for cp in copies:
            cp.wait()

    for cp in tail_copies:
        cp.wait()
    plsc.subcore_barrier()
    d_copies = []
    for c in range(3):
        d_copies.append(pltpu.async_copy(
            d_sp.at[pl.ds(c * BP, BP)], d_v.at[pl.ds(c * DP, BP)], sem))
    for cp in d_copies:
        cp.wait()

    iota = lax.iota(jnp.int32, NLANE)
    nmax = jnp.full((NLANE,), N - 1, jnp.int32)
    zero = jnp.zeros((NLANE,), jnp.int32)
    pc = [c * DP + DELTA_D[c] for c in range(3)]

    def group(g, acc):
        vb = g * NLANE
        vglob = base + vb + iota
        vmin = jnp.minimum(vglob, nmax)
        s0 = plsc.load_gather(d_v, [vmin + pc[0]])
        s1 = plsc.load_gather(d_v, [vmin + pc[1]])
        s2 = plsc.load_gather(d_v, [vmin + pc[2]])
        a0 = jnp.zeros((NLANE,), jnp.float32)
        a1 = jnp.zeros((NLANE,), jnp.float32)
        a2 = jnp.zeros((NLANE,), jnp.float32)
        for j in range(K):
            nb = plsc.load_gather(idx_v, [vb + iota + (j * CP + DELTA_I[j])])
            nb = jnp.minimum(jnp.maximum(nb, zero), nmax)
            a0 = a0 + plsc.load_gather(d_v, [nb + pc[0]])
            a1 = a1 + plsc.load_gather(d_v, [nb + pc[1]])
            a2 = a2 + plsc.load_gather(d_v, [nb + pc[2]])
        rw = 1.0 / aw_v[pl.ds(vb, NLANE)]
        r0 = s0 - a0 * rw
        r1 = s1 - a1 * rw
        r2 = s2 - a2 * rw
        lwt = lw_v[pl.ds(vb, NLANE)]
        term = (r0 * r0 + r1 * r1 + r2 * r2) * lwt
        term = jnp.where(vglob < N, term, 0.0)
        return acc + term

    acc = lax.fori_loop(0, GROUPS, group, jnp.zeros((NLANE,), jnp.float32))
    acc_v[...] = acc
    pltpu.sync_copy(acc_v, out_hbm.at[pl.ds(wid * NLANE, NLANE)])


_sc_call = pl.kernel(
    _sc_body,
    out_type=jax.ShapeDtypeStruct((NTILES * NLANE,), jnp.float32),
    mesh=plsc.VectorSubcoreMesh(core_axis_name="c", subcore_axis_name="s"),
    compiler_params=pltpu.CompilerParams(
        needs_layout_passes=False, use_tc_tiling_on_sc=False),
    scratch_types=[
        pltpu.VMEM((3 * DP,), jnp.float32),
        pltpu.VMEM((K * CP,), jnp.int32),
        pltpu.VMEM((CHUNK,), jnp.float32),
        pltpu.VMEM((CHUNK,), jnp.float32),
        pltpu.VMEM((NLANE,), jnp.float32),
        pltpu.VMEM_SHARED((3 * BP,), jnp.float32),
        pltpu.SemaphoreType.DMA,
    ],
)


def kernel(v_1, v_2, adjacency_idx, adjacency_w, laplace_w):
    d_flat = (v_1 - v_2).T.reshape(3 * N)
    idx_flat = adjacency_idx.astype(jnp.int32).T.reshape(K * N)
    partials = _sc_call(
        d_flat, idx_flat, adjacency_w.reshape(N), laplace_w.reshape(N)
    ).reshape(NTILES, NLANE)
    out = pl.pallas_call(
        _final_body,
        out_shape=jax.ShapeDtypeStruct((1, 1), jnp.float32),
    )(partials)
    return out.reshape(())
